# baseline (speedup 1.0000x reference)
import functools

import jax
import jax.numpy as jnp
from jax import lax
from jax.experimental import pallas as pl
from jax.experimental.pallas import tpu as pltpu
from jax.experimental.pallas import tpu_sc as plsc


def _build_sc_kernel(V, D, B, L):
    info = plsc.get_sparse_core_info()
    NC, NS, LN = info.num_cores, info.num_subcores, info.num_lanes
    NW = NC * NS
    assert B % NW == 0 and D % LN == 0
    b_per_w = B // NW
    CB = 16
    n_chunks = b_per_w // CB
    rows_per_chunk = CB * L
    n_vregs = D // LN
    GR = 32

    mesh = plsc.VectorSubcoreMesh(core_axis_name="c", subcore_axis_name="s")

    @functools.partial(
        pl.kernel,
        mesh=mesh,
        out_type=jax.ShapeDtypeStruct((NW, LN), jnp.float32),
        scratch_types=[
            pltpu.VMEM((b_per_w, L), jnp.int32),
            pltpu.VMEM((b_per_w * L + 4 * LN,), jnp.int32),
            pltpu.VMEM((2 * LN,), jnp.int32),
            pltpu.VMEM((b_per_w + LN,), jnp.int32),
            pltpu.VMEM((b_per_w + LN,), jnp.float32),
            pltpu.VMEM((b_per_w,), jnp.int32),
            pltpu.VMEM((rows_per_chunk, D), jnp.float32),
            pltpu.VMEM((rows_per_chunk, D), jnp.float32),
            pltpu.VMEM((CB, D), jnp.float32),
            pltpu.VMEM((CB, D), jnp.float32),
            pltpu.VMEM((LN,), jnp.float32),
            pltpu.SemaphoreType.DMA,
            pltpu.SemaphoreType.DMA,
        ],
    )
    def sc_kernel(emb_hbm, cw_hbm, len_hbm, inv_hbm, mwe_hbm, out_hbm,
                  idx_v, cidx_v, cnts_v, len_v, invf_v, midx_v,
                  rows_a, rows_b, tgt_a, tgt_b,
                  out_v, sem_a, sem_b):
        wid = lax.axis_index("s") * NC + lax.axis_index("c")
        base_b = wid * b_per_w

        c1 = pltpu.async_copy(cw_hbm.at[pl.ds(base_b, b_per_w)], idx_v,
                              sem_a)
        c2 = pltpu.async_copy(len_hbm.at[pl.ds(base_b, b_per_w)],
                              len_v.at[pl.ds(0, b_per_w)], sem_a)
        c3 = pltpu.async_copy(mwe_hbm.at[pl.ds(base_b, b_per_w)], midx_v,
                              sem_a)
        c4 = pltpu.async_copy(inv_hbm.at[pl.ds(base_b, b_per_w)],
                              invf_v.at[pl.ds(0, b_per_w)], sem_a)
        c1.wait()
        c2.wait()
        c3.wait()
        c4.wait()

        zeros = jnp.zeros((LN,), jnp.float32)
        izeros = jnp.zeros((LN,), jnp.int32)
        iota = lax.broadcasted_iota(jnp.int32, (LN,), 0)
        sq = (zeros,) * n_vregs

        def compact_chunk(cb, counts_vec):
            cstart = cb * rows_per_chunk

            def pack_b(b, pos, _cb=cb):
                ln = len_v[pl.ds(_cb * CB + b, LN)][0]
                row = _cb * CB + b
                v0 = idx_v[row, pl.ds(0, LN)]
                v1 = idx_v[row, pl.ds(L - LN, LN)]
                cidx_v[pl.ds(pos, LN)] = v0
                cidx_v[pl.ds(pos + (L - LN), LN)] = v1
                return pos + ln

            pos = lax.fori_loop(0, CB, pack_b, jnp.int32(cstart))
            pad = iota + (wid * n_chunks + cb) * 2 * LN
            cidx_v[pl.ds(pos, LN)] = pad
            cidx_v[pl.ds(pos + LN, LN)] = pad + LN
            k_cb = (pos - cstart + (GR - 1)) // GR
            counts_vec = jnp.where(iota == cb, jnp.full((LN,), k_cb),
                                   counts_vec)
            return counts_vec, k_cb

        bufs = ((rows_a, tgt_a, sem_a), (rows_b, tgt_b, sem_b))

        n_gr = rows_per_chunk // GR

        def fire(cb, par, kk=None):
            rows_v, tgt_v, sem = bufs[par]
            if kk is None:
                kk = cnts_v[pl.ds(cb, LN)][0]
            for g in range(n_gr):
                @pl.when(g < kk)
                def _(_g=g):
                    src = pl.multiple_of(cb * rows_per_chunk + _g * GR, GR)
                    pltpu.async_copy(
                        emb_hbm.at[cidx_v.at[pl.ds(src, GR)]],
                        rows_v.at[pl.ds(_g * GR, GR)], sem)
            pltpu.async_copy(
                emb_hbm.at[midx_v.at[pl.ds(pl.multiple_of(cb * CB, CB), CB)]],
                tgt_v, sem)

        def drain(cb, par):
            rows_v, tgt_v, sem = bufs[par]
            kk = cnts_v[pl.ds(cb, LN)][0]
            for g in range(n_gr):
                @pl.when(g < kk)
                def _(_g=g):
                    pltpu.make_async_copy(
                        emb_hbm.at[cidx_v.at[pl.ds(0, GR)]],
                        rows_v.at[pl.ds(_g * GR, GR)], sem).wait()
            pltpu.make_async_copy(
                emb_hbm.at[midx_v.at[pl.ds(0, CB)]], tgt_v, sem).wait()

        def compute(cb, par, sq):
            rows_v, tgt_v, _ = bufs[par]

            def b_body(b, carry, _cb=cb):
                sq = carry[:n_vregs]
                rbase = carry[n_vregs]
                ln = len_v[pl.ds(_cb * CB + b, LN)][0]
                inv = jnp.full((LN,),
                               invf_v[pl.ds(_cb * CB + b, LN)][0])

                def l_body(l, accs, _rb=rbase):
                    r = _rb + l
                    return tuple(accs[j] + rows_v[r, pl.ds(j * LN, LN)]
                                 for j in range(n_vregs))

                accs = lax.fori_loop(0, ln, l_body, (zeros,) * n_vregs)
                out = []
                for j in range(n_vregs):
                    diff = accs[j] * inv - tgt_v[b, pl.ds(j * LN, LN)]
                    out.append(sq[j] + diff * diff)
                return tuple(out) + (rbase + ln,)

            carry = lax.fori_loop(0, CB, b_body, sq + (jnp.int32(0),))
            return carry[:n_vregs]

        counts_vec = izeros
        counts_vec, k0 = compact_chunk(0, counts_vec)
        counts_vec, k1 = compact_chunk(1, counts_vec)
        fire(0, 0, kk=k0)
        fire(1, 1, kk=k1)
        for cb in range(2, n_chunks):
            counts_vec, _ = compact_chunk(cb, counts_vec)
        cnts_v[pl.ds(0, LN)] = counts_vec
        n_pairs = n_chunks // 2

        def pair_body(k, sq):
            c = 2 * k
            drain(c, 0)
            sq = compute(c, 0, sq)

            @pl.when(k < n_pairs - 1)
            def _():
                fire(c + 2, 0)

            drain(c + 1, 1)
            sq = compute(c + 1, 1, sq)

            @pl.when(k < n_pairs - 1)
            def _():
                fire(c + 3, 1)

            return sq

        sq = lax.fori_loop(0, n_pairs, pair_body, sq)

        total = sq[0]
        for j in range(1, n_vregs):
            total = total + sq[j]
        out_v[...] = total
        pltpu.sync_copy(out_v, out_hbm.at[wid])

    return sc_kernel


def kernel(emb, center_words, center_words_len, mwe_words):
    V, D = emb.shape
    B, L = center_words.shape
    sc = _build_sc_kernel(V, D, B, L)
    inv_len = 1.0 / center_words_len.astype(jnp.float32)
    partials = sc(emb, center_words, center_words_len, inv_len, mwe_words)
    return jnp.sum(partials) / jnp.float32(B * D)


# speedup vs baseline: 1.0181x; 1.0181x over previous
import functools

import jax
import jax.numpy as jnp
from jax import lax
from jax.experimental import pallas as pl
from jax.experimental.pallas import tpu as pltpu
from jax.experimental.pallas import tpu_sc as plsc


def _build_sc_kernel(V, D, B, L):
    info = plsc.get_sparse_core_info()
    NC, NS, LN = info.num_cores, info.num_subcores, info.num_lanes
    NW = NC * NS
    assert B % NW == 0 and D % LN == 0
    b_per_w = B // NW
    CB = 16
    n_chunks = b_per_w // CB
    rows_per_chunk = CB * L
    n_vregs = D // LN
    GR = 32

    mesh = plsc.VectorSubcoreMesh(core_axis_name="c", subcore_axis_name="s")

    @functools.partial(
        pl.kernel,
        mesh=mesh,
        out_type=jax.ShapeDtypeStruct((NW, LN), jnp.float32),
        scratch_types=[
            pltpu.VMEM((b_per_w, L), jnp.int32),
            pltpu.VMEM((b_per_w * L + 4 * LN,), jnp.int32),
            pltpu.VMEM((2 * LN,), jnp.int32),
            pltpu.VMEM((b_per_w + LN,), jnp.int32),
            pltpu.VMEM((b_per_w,), jnp.int32),
            pltpu.VMEM((rows_per_chunk, D), jnp.float32),
            pltpu.VMEM((rows_per_chunk, D), jnp.float32),
            pltpu.VMEM((CB, D), jnp.float32),
            pltpu.VMEM((CB, D), jnp.float32),
            pltpu.VMEM((LN,), jnp.float32),
            pltpu.SemaphoreType.DMA,
            pltpu.SemaphoreType.DMA,
        ],
    )
    def sc_kernel(emb_hbm, cw_hbm, len_hbm, mwe_hbm, out_hbm,
                  idx_v, cidx_v, cnts_v, len_v, midx_v,
                  rows_a, rows_b, tgt_a, tgt_b,
                  out_v, sem_a, sem_b):
        wid = lax.axis_index("s") * NC + lax.axis_index("c")
        base_b = wid * b_per_w

        c1 = pltpu.async_copy(cw_hbm.at[pl.ds(base_b, b_per_w)], idx_v,
                              sem_a)
        c2 = pltpu.async_copy(len_hbm.at[pl.ds(base_b, b_per_w)],
                              len_v.at[pl.ds(0, b_per_w)], sem_a)
        c3 = pltpu.async_copy(mwe_hbm.at[pl.ds(base_b, b_per_w)], midx_v,
                              sem_a)
        c1.wait()
        c2.wait()
        c3.wait()

        zeros = jnp.zeros((LN,), jnp.float32)
        izeros = jnp.zeros((LN,), jnp.int32)
        iota = lax.broadcasted_iota(jnp.int32, (LN,), 0)
        sq = (zeros,) * n_vregs

        def compact_chunk(cb, counts_vec):
            cstart = cb * rows_per_chunk

            def pack_b(b, pos, _cb=cb):
                ln = len_v[pl.ds(_cb * CB + b, LN)][0]
                row = _cb * CB + b
                v0 = idx_v[row, pl.ds(0, LN)]
                v1 = idx_v[row, pl.ds(L - LN, LN)]
                cidx_v[pl.ds(pos, LN)] = v0
                cidx_v[pl.ds(pos + (L - LN), LN)] = v1
                return pos + ln

            pos = lax.fori_loop(0, CB, pack_b, jnp.int32(cstart))
            pad = iota + (wid * n_chunks + cb) * 2 * LN
            cidx_v[pl.ds(pos, LN)] = pad
            cidx_v[pl.ds(pos + LN, LN)] = pad + LN
            k_cb = (pos - cstart + (GR - 1)) // GR
            counts_vec = jnp.where(iota == cb, jnp.full((LN,), k_cb),
                                   counts_vec)
            return counts_vec, k_cb

        bufs = ((rows_a, tgt_a, sem_a), (rows_b, tgt_b, sem_b))

        n_gr = rows_per_chunk // GR

        def fire(cb, par, kk=None):
            rows_v, tgt_v, sem = bufs[par]
            if kk is None:
                kk = cnts_v[pl.ds(cb, LN)][0]
            for g in range(n_gr):
                @pl.when(g < kk)
                def _(_g=g):
                    src = pl.multiple_of(cb * rows_per_chunk + _g * GR, GR)
                    pltpu.async_copy(
                        emb_hbm.at[cidx_v.at[pl.ds(src, GR)]],
                        rows_v.at[pl.ds(_g * GR, GR)], sem)
            pltpu.async_copy(
                emb_hbm.at[midx_v.at[pl.ds(pl.multiple_of(cb * CB, CB), CB)]],
                tgt_v, sem)

        def drain(cb, par):
            rows_v, tgt_v, sem = bufs[par]
            kk = cnts_v[pl.ds(cb, LN)][0]
            for g in range(n_gr):
                @pl.when(g < kk)
                def _(_g=g):
                    pltpu.make_async_copy(
                        emb_hbm.at[cidx_v.at[pl.ds(0, GR)]],
                        rows_v.at[pl.ds(_g * GR, GR)], sem).wait()
            pltpu.make_async_copy(
                emb_hbm.at[midx_v.at[pl.ds(0, CB)]], tgt_v, sem).wait()

        def compute(cb, par, sq):
            rows_v, tgt_v, _ = bufs[par]

            def b_body(b, carry, _cb=cb):
                sq = carry[:n_vregs]
                rbase = carry[n_vregs]
                ln = len_v[pl.ds(_cb * CB + b, LN)][0]
                inv = jnp.full((LN,), 1.0, jnp.float32) / jnp.full(
                    (LN,), ln).astype(jnp.float32)

                def l_body(l, accs, _rb=rbase):
                    r = _rb + l
                    return tuple(accs[j] + rows_v[r, pl.ds(j * LN, LN)]
                                 for j in range(n_vregs))

                accs = lax.fori_loop(0, ln, l_body, (zeros,) * n_vregs)
                out = []
                for j in range(n_vregs):
                    diff = accs[j] * inv - tgt_v[b, pl.ds(j * LN, LN)]
                    out.append(sq[j] + diff * diff)
                return tuple(out) + (rbase + ln,)

            carry = lax.fori_loop(0, CB, b_body, sq + (jnp.int32(0),))
            return carry[:n_vregs]

        counts_vec = izeros
        counts_vec, k0 = compact_chunk(0, counts_vec)
        counts_vec, k1 = compact_chunk(1, counts_vec)
        fire(0, 0, kk=k0)
        fire(1, 1, kk=k1)
        for cb in range(2, n_chunks):
            counts_vec, _ = compact_chunk(cb, counts_vec)
        cnts_v[pl.ds(0, LN)] = counts_vec
        n_pairs = n_chunks // 2

        def pair_body(k, sq):
            c = 2 * k
            drain(c, 0)
            sq = compute(c, 0, sq)

            @pl.when(k < n_pairs - 1)
            def _():
                fire(c + 2, 0)

            drain(c + 1, 1)
            sq = compute(c + 1, 1, sq)

            @pl.when(k < n_pairs - 1)
            def _():
                fire(c + 3, 1)

            return sq

        sq = lax.fori_loop(0, n_pairs, pair_body, sq)

        total = sq[0]
        for j in range(1, n_vregs):
            total = total + sq[j]
        out_v[...] = total
        pltpu.sync_copy(out_v, out_hbm.at[wid])

    return sc_kernel


def kernel(emb, center_words, center_words_len, mwe_words):
    V, D = emb.shape
    B, L = center_words.shape
    sc = _build_sc_kernel(V, D, B, L)
    partials = sc(emb, center_words, center_words_len, mwe_words)
    return jnp.sum(partials) / jnp.float32(B * D)
